# baseline (device time: 18248 ns/iter reference)
import jax
import jax.numpy as jnp
from jax import lax
from jax.experimental import pallas as pl
from jax.experimental.pallas import tpu as pltpu


def kernel(ids, E):
    t = ids.shape[0]
    v, d = E.shape
    hp = t // 2

    def body(ids_ref, e_ref, out_ref,
             xsend_ref, xrecv_ref, ysend_ref, yrecv_ref, sems):
        my_x = lax.axis_index("x")
        my_y = lax.axis_index("y")
        x_peer = (1 - my_x, my_y)
        y_peer = (my_x, 1 - my_y)

        barrier_sem = pltpu.get_barrier_semaphore()
        for nbr in (x_peer, y_peer):
            pl.semaphore_signal(
                barrier_sem, inc=1,
                device_id=nbr, device_id_type=pl.DeviceIdType.MESH,
            )
        pl.semaphore_wait(barrier_sem, 2)

        lid = ids_ref[pl.ds(my_y * hp, hp)] - my_x * v
        col = lax.broadcasted_iota(jnp.int32, (hp, v), 1)
        onehot = (col == lid[:, None]).astype(jnp.bfloat16)
        e_bf = e_ref[:, :].astype(jnp.bfloat16)
        xsend_ref[:, :] = jnp.dot(
            onehot, e_bf, preferred_element_type=jnp.float32
        ).astype(jnp.bfloat16)

        x_rdma = pltpu.make_async_remote_copy(
            src_ref=xsend_ref, dst_ref=xrecv_ref,
            send_sem=sems.at[0], recv_sem=sems.at[1],
            device_id=x_peer, device_id_type=pl.DeviceIdType.MESH,
        )
        x_rdma.start()
        x_rdma.wait()
        red = xsend_ref[:, :] + xrecv_ref[:, :]
        ysend_ref[:, :] = red
        out_ref[pl.ds(my_y * hp, hp), :] = red.astype(jnp.float32)

        y_rdma = pltpu.make_async_remote_copy(
            src_ref=ysend_ref, dst_ref=yrecv_ref,
            send_sem=sems.at[2], recv_sem=sems.at[3],
            device_id=y_peer, device_id_type=pl.DeviceIdType.MESH,
        )
        y_rdma.start()
        y_rdma.wait()
        out_ref[pl.ds((1 - my_y) * hp, hp), :] = (
            yrecv_ref[:, :].astype(jnp.float32)
        )

    return pl.pallas_call(
        body,
        out_shape=jax.ShapeDtypeStruct((t, d), jnp.float32),
        in_specs=[
            pl.BlockSpec(memory_space=pltpu.VMEM),
            pl.BlockSpec(memory_space=pltpu.VMEM),
        ],
        out_specs=pl.BlockSpec(memory_space=pltpu.VMEM),
        scratch_shapes=[
            pltpu.VMEM((hp, d), jnp.bfloat16),
            pltpu.VMEM((hp, d), jnp.bfloat16),
            pltpu.VMEM((hp, d), jnp.bfloat16),
            pltpu.VMEM((hp, d), jnp.bfloat16),
            pltpu.SemaphoreType.DMA((4,)),
        ],
        compiler_params=pltpu.CompilerParams(collective_id=0),
    )(ids, E)


# device time: 16131 ns/iter; 1.1312x vs baseline; 1.1312x over previous
import jax
import jax.numpy as jnp
from jax import lax
from jax.experimental import pallas as pl
from jax.experimental.pallas import tpu as pltpu

NC = 4


def kernel(ids, E):
    t = ids.shape[0]
    v, d = E.shape
    hp = t // 2
    ck = hp // NC

    def body(ids_ref, e_ref, out_ref, xsend, xrecv,
             xs_sems, xr_sems, ys_sems, yr_sems):
        my_x = lax.axis_index("x")
        my_y = lax.axis_index("y")
        x_peer = (1 - my_x, my_y)
        y_peer = (my_x, 1 - my_y)
        base = my_y * hp
        obase = (1 - my_y) * hp

        barrier_sem = pltpu.get_barrier_semaphore()
        for nbr in (x_peer, y_peer):
            pl.semaphore_signal(
                barrier_sem, inc=1,
                device_id=nbr, device_id_type=pl.DeviceIdType.MESH,
            )

        x_rdmas = []
        for c in range(NC):
            lid = ids_ref[pl.ds(base + c * ck, ck), :] - my_x * v
            col = lax.broadcasted_iota(jnp.int32, (ck, v), 1)
            oh = (col == jnp.broadcast_to(lid, (ck, v))).astype(jnp.bfloat16)
            xsend[c, :, :] = lax.dot_general(
                oh, e_ref[:, :], (((1,), (0,)), ((), ())),
                preferred_element_type=jnp.float32,
            ).astype(jnp.bfloat16)
            if c == 0:
                pl.semaphore_wait(barrier_sem, 2)
            rdma = pltpu.make_async_remote_copy(
                src_ref=xsend.at[c], dst_ref=xrecv.at[c],
                send_sem=xs_sems.at[c], recv_sem=xr_sems.at[c],
                device_id=x_peer, device_id_type=pl.DeviceIdType.MESH,
            )
            rdma.start()
            x_rdmas.append(rdma)

        y_rdmas = []
        for c in range(NC):
            x_rdmas[c].wait_recv()
            out_ref[pl.ds(base + c * ck, ck), :] = (
                xsend[c, :, :] + xrecv[c, :, :]
            )
            rdma = pltpu.make_async_remote_copy(
                src_ref=out_ref.at[pl.ds(base + c * ck, ck)],
                dst_ref=out_ref.at[pl.ds(base + c * ck, ck)],
                send_sem=ys_sems.at[c], recv_sem=yr_sems.at[c],
                device_id=y_peer, device_id_type=pl.DeviceIdType.MESH,
            )
            rdma.start()
            y_rdmas.append(rdma)

        for c in range(NC):
            recv = pltpu.make_async_remote_copy(
                src_ref=out_ref.at[pl.ds(obase + c * ck, ck)],
                dst_ref=out_ref.at[pl.ds(obase + c * ck, ck)],
                send_sem=ys_sems.at[c], recv_sem=yr_sems.at[c],
                device_id=y_peer, device_id_type=pl.DeviceIdType.MESH,
            )
            recv.wait_recv()
        for c in range(NC):
            x_rdmas[c].wait_send()
            y_rdmas[c].wait_send()

    return pl.pallas_call(
        body,
        out_shape=jax.ShapeDtypeStruct((t, d), jnp.bfloat16),
        in_specs=[
            pl.BlockSpec(memory_space=pltpu.VMEM),
            pl.BlockSpec(memory_space=pltpu.VMEM),
        ],
        out_specs=pl.BlockSpec(memory_space=pltpu.VMEM),
        scratch_shapes=[
            pltpu.VMEM((NC, ck, d), jnp.bfloat16),
            pltpu.VMEM((NC, ck, d), jnp.bfloat16),
            pltpu.SemaphoreType.DMA((NC,)),
            pltpu.SemaphoreType.DMA((NC,)),
            pltpu.SemaphoreType.DMA((NC,)),
            pltpu.SemaphoreType.DMA((NC,)),
        ],
        compiler_params=pltpu.CompilerParams(collective_id=0),
    )(ids.reshape(t, 1), E)


# device time: 15609 ns/iter; 1.1691x vs baseline; 1.0334x over previous
import jax
import jax.numpy as jnp
from jax import lax
from jax.experimental import pallas as pl
from jax.experimental.pallas import tpu as pltpu

NC = 8


def kernel(ids, E):
    t = ids.shape[0]
    v, d = E.shape
    hp = t // 2
    ck = hp // NC

    def body(ids_ref, e_ref, out_ref, xsend, xrecv,
             xs_sems, xr_sems, ys_sems, yr_sems):
        my_x = lax.axis_index("x")
        my_y = lax.axis_index("y")
        x_peer = (1 - my_x, my_y)
        y_peer = (my_x, 1 - my_y)
        base = my_y * hp
        obase = (1 - my_y) * hp

        barrier_sem = pltpu.get_barrier_semaphore()
        for nbr in (x_peer, y_peer):
            pl.semaphore_signal(
                barrier_sem, inc=1,
                device_id=nbr, device_id_type=pl.DeviceIdType.MESH,
            )

        lid = (ids_ref[pl.ds(base, hp)] - my_x * v)[:, None]
        col = lax.broadcasted_iota(jnp.int32, (hp, v), 1)
        oh = (col == jnp.broadcast_to(lid, (hp, v))).astype(jnp.bfloat16)
        r = lax.dot_general(
            oh, e_ref[:, :], (((1,), (0,)), ((), ())),
            preferred_element_type=jnp.float32,
        ).astype(jnp.bfloat16)

        x_rdmas = []
        for c in range(NC):
            xsend[c, :, :] = r[c * ck:(c + 1) * ck, :]
            if c == 0:
                pl.semaphore_wait(barrier_sem, 2)
            rdma = pltpu.make_async_remote_copy(
                src_ref=xsend.at[c], dst_ref=xrecv.at[c],
                send_sem=xs_sems.at[c], recv_sem=xr_sems.at[c],
                device_id=x_peer, device_id_type=pl.DeviceIdType.MESH,
            )
            rdma.start()
            x_rdmas.append(rdma)

        y_rdmas = []
        for c in range(NC):
            x_rdmas[c].wait_recv()
            out_ref[pl.ds(base + c * ck, ck), :] = (
                xsend[c, :, :] + xrecv[c, :, :]
            )
            rdma = pltpu.make_async_remote_copy(
                src_ref=out_ref.at[pl.ds(base + c * ck, ck)],
                dst_ref=out_ref.at[pl.ds(base + c * ck, ck)],
                send_sem=ys_sems.at[c], recv_sem=yr_sems.at[c],
                device_id=y_peer, device_id_type=pl.DeviceIdType.MESH,
            )
            rdma.start()
            y_rdmas.append(rdma)

        for c in range(NC):
            recv = pltpu.make_async_remote_copy(
                src_ref=out_ref.at[pl.ds(obase + c * ck, ck)],
                dst_ref=out_ref.at[pl.ds(obase + c * ck, ck)],
                send_sem=ys_sems.at[c], recv_sem=yr_sems.at[c],
                device_id=y_peer, device_id_type=pl.DeviceIdType.MESH,
            )
            recv.wait_recv()
        for c in range(NC):
            x_rdmas[c].wait_send()
            y_rdmas[c].wait_send()

    return pl.pallas_call(
        body,
        out_shape=jax.ShapeDtypeStruct((t, d), jnp.bfloat16),
        in_specs=[
            pl.BlockSpec(memory_space=pltpu.VMEM),
            pl.BlockSpec(memory_space=pltpu.VMEM),
        ],
        out_specs=pl.BlockSpec(memory_space=pltpu.VMEM),
        scratch_shapes=[
            pltpu.VMEM((NC, ck, d), jnp.bfloat16),
            pltpu.VMEM((NC, ck, d), jnp.bfloat16),
            pltpu.SemaphoreType.DMA((NC,)),
            pltpu.SemaphoreType.DMA((NC,)),
            pltpu.SemaphoreType.DMA((NC,)),
            pltpu.SemaphoreType.DMA((NC,)),
        ],
        compiler_params=pltpu.CompilerParams(collective_id=0),
    )(ids, E)
